# baseline (speedup 1.0000x reference)
import jax
import jax.numpy as jnp
from jax import lax
from jax.experimental import pallas as pl
from jax.experimental.pallas import tpu as pltpu
from jax.experimental.pallas import tpu_sc as plsc

_BATCH, _NP, _D = 64, 576, 768
_NC, _NS = 2, 16
_NW = _NC * _NS
_BPW = _BATCH // _NW
_C = 64
_NCHUNK = _NP // _C
_T = _BPW * _NCHUNK
_VPR = _D // 16


def _sc_body(flat_hbm, pos_hbm, out_hbm,
             spmem_pos, posbuf, buf, sem_pre, sem_in, sem_out):
    c = lax.axis_index("c")
    s = lax.axis_index("s")
    w = c * _NS + s
    @pl.when(s < 8)
    def _stage():
        pltpu.sync_copy(pos_hbm.at[pl.ds(s * 72, 72)], spmem_pos.at[pl.ds(s * 72, 72)])

    plsc.subcore_barrier()

    def step(t, carry):
        row0 = w * _BPW * _NP + t * _C
        p0 = (t % _NCHUNK) * _C
        cp_pre = pltpu.async_copy(spmem_pos.at[pl.ds(p0, _C)], posbuf, sem_pre)
        cp_in = pltpu.async_copy(flat_hbm.at[pl.ds(row0, _C)], buf, sem_in)
        cp_pre.wait()
        cp_in.wait()

        def row_add(i, carry2):
            for j in range(_VPR):
                plsc.addupdate(buf.at[i, pl.ds(j * 16, 16)],
                               posbuf[i, pl.ds(j * 16, 16)])
            return carry2

        lax.fori_loop(0, _C, row_add, 0)
        pltpu.async_copy(buf, out_hbm.at[pl.ds(row0, _C)], sem_out).wait()
        return carry

    lax.fori_loop(0, _T, step, 0)


def kernel(patches, pos_table):
    flat = patches.reshape(_BATCH * _NP, _D)
    mesh = plsc.VectorSubcoreMesh(core_axis_name="c", subcore_axis_name="s")
    out = pl.kernel(
        _sc_body,
        out_type=jax.ShapeDtypeStruct((_BATCH * _NP, _D), jnp.float32),
        mesh=mesh,
        scratch_types=[
            pltpu.VMEM_SHARED((_NP, _D), jnp.float32),
            pltpu.VMEM((_C, _D), jnp.float32),
            pltpu.VMEM((_C, _D), jnp.float32),
            pltpu.SemaphoreType.DMA,
            pltpu.SemaphoreType.DMA,
            pltpu.SemaphoreType.DMA,
        ],
    )(flat, pos_table)
    return out.reshape(_BATCH, _NP, _D)
